# baseline (speedup 1.0000x reference)
import functools

import jax
import jax.numpy as jnp
from jax import lax
from jax.experimental import pallas as pl
from jax.experimental.pallas import tpu as pltpu
from jax.experimental.pallas import tpu_sc as plsc

B = 16384
NNUM = 13
NF = 26
V = 100000
D = 16
H1 = 128
H2 = 64

NC, NS = 2, 16
NW = NC * NS
FPLANES = (8, 8, 8, 2)

VB = 2048


def _tr_body(et_ref, out_ref):
    out_ref[...] = jnp.transpose(et_ref[...], (1, 0))


@functools.cache
def _make_transpose(fk):
    return pl.pallas_call(
        _tr_body,
        grid=((V + VB - 1) // VB,),
        in_specs=[pl.BlockSpec((fk * D, VB), lambda j: (0, j))],
        out_specs=pl.BlockSpec((VB, fk * D), lambda j: (j, 0)),
        out_shape=jax.ShapeDtypeStruct((V, fk * D), jnp.float32),
    )


@functools.cache
def _make_sc_gather(fk):
    mesh = plsc.VectorSubcoreMesh(
        core_axis_name="c", subcore_axis_name="s", num_cores=NC, num_subcores=NS
    )
    rows = B * fk
    rpw = rows // NW

    @functools.partial(
        pl.kernel,
        out_type=jax.ShapeDtypeStruct((rows, D), jnp.float32),
        mesh=mesh,
        scratch_types=[
            pltpu.VMEM((rpw,), jnp.int32),
            pltpu.VMEM((rpw, D), jnp.float32),
            pltpu.SemaphoreType.DMA,
        ],
        compiler_params=pltpu.CompilerParams(use_tc_tiling_on_sc=False),
    )
    def _sc_gather(idx_hbm, table_hbm, out_hbm, idx_v, rows_v, sem):
        wid = lax.axis_index("s") * NC + lax.axis_index("c")
        base = wid * rpw
        pltpu.sync_copy(idx_hbm.at[pl.ds(base, rpw)], idx_v)
        pltpu.async_copy(table_hbm.at[idx_v], rows_v, sem).wait()
        pltpu.sync_copy(rows_v, out_hbm.at[pl.ds(base, rpw)])

    return _sc_gather


BLK = 2048


def _mlp_body(xn_ref, xe0_ref, xe1_ref, xe2_ref, xe3_ref, w1n_ref, w10_ref,
              w11_ref, w12_ref, w13_ref, b1_ref, w2_ref, b2_ref, w3_ref,
              b3_ref, out_ref):
    h1 = jnp.dot(xe0_ref[...], w10_ref[...], preferred_element_type=jnp.float32)
    h1 += jnp.dot(xe1_ref[...], w11_ref[...], preferred_element_type=jnp.float32)
    h1 += jnp.dot(xe2_ref[...], w12_ref[...], preferred_element_type=jnp.float32)
    h1 += jnp.dot(xe3_ref[...], w13_ref[...], preferred_element_type=jnp.float32)
    h1 += jnp.dot(xn_ref[...], w1n_ref[...], preferred_element_type=jnp.float32)
    h1 = jnp.maximum(h1 + b1_ref[...], 0.0)
    h2 = jnp.maximum(
        jnp.dot(h1, w2_ref[...], preferred_element_type=jnp.float32) + b2_ref[...],
        0.0,
    )
    out_ref[...] = (
        jnp.dot(h2, w3_ref[...], preferred_element_type=jnp.float32) + b3_ref[...]
    )


def _blk(r, c, im):
    return pl.BlockSpec((r, c), im)


_mlp = pl.pallas_call(
    _mlp_body,
    grid=(B // BLK,),
    in_specs=[
        _blk(BLK, NNUM, lambda i: (i, 0)),
        _blk(BLK, FPLANES[0] * D, lambda i: (i, 0)),
        _blk(BLK, FPLANES[1] * D, lambda i: (i, 0)),
        _blk(BLK, FPLANES[2] * D, lambda i: (i, 0)),
        _blk(BLK, FPLANES[3] * D, lambda i: (i, 0)),
        _blk(NNUM, H1, lambda i: (0, 0)),
        _blk(FPLANES[0] * D, H1, lambda i: (0, 0)),
        _blk(FPLANES[1] * D, H1, lambda i: (0, 0)),
        _blk(FPLANES[2] * D, H1, lambda i: (0, 0)),
        _blk(FPLANES[3] * D, H1, lambda i: (0, 0)),
        _blk(1, H1, lambda i: (0, 0)),
        _blk(H1, H2, lambda i: (0, 0)),
        _blk(1, H2, lambda i: (0, 0)),
        _blk(H2, 1, lambda i: (0, 0)),
        _blk(1, 1, lambda i: (0, 0)),
    ],
    out_specs=pl.BlockSpec((BLK, 1), lambda i: (i, 0)),
    out_shape=jax.ShapeDtypeStruct((B, 1), jnp.float32),
)


def kernel(x_num, x_cat, E, W1, b1, W2, b2, W3, b3):
    et = jnp.transpose(E, (0, 2, 1)).reshape(NF * D, V)

    xes = []
    w1s = []
    f0 = 0
    for fk in FPLANES:
        table = _make_transpose(fk)(et[f0 * D:(f0 + fk) * D])
        j = jnp.arange(fk, dtype=jnp.int32)
        idx = (x_cat[:, f0:f0 + fk] * fk + j[None, :]).reshape(B * fk)
        emb = _make_sc_gather(fk)(idx, table.reshape(V * fk, D))
        xes.append(emb.reshape(B, fk * D))
        w1s.append(W1[NNUM + f0 * D: NNUM + (f0 + fk) * D])
        f0 += fk

    return _mlp(
        x_num, *xes,
        W1[:NNUM], *w1s, b1[None, :],
        W2, b2[None, :],
        W3, b3[None, :],
    )
